# baseline (speedup 1.0000x reference)
import functools

import jax
import jax.numpy as jnp
from jax import lax
from jax.experimental import pallas as pl
from jax.experimental.pallas import tpu as pltpu
from jax.experimental.pallas import tpu_sc as plsc

N_CAMS = 16384
N_VIEWS = 256
_NC = 2
_NS = 16
_NW = _NC * _NS
_CPW = N_CAMS // _NW
_L = 16
_GROUPS = _CPW // _L


def _rsqrt16(s):
    i = lax.bitcast_convert_type(s, jnp.int32)
    i = jnp.int32(0x5F3759DF) - lax.shift_right_arithmetic(i, 1)
    y = lax.bitcast_convert_type(i, jnp.float32)
    half_s = 0.5 * s
    for _ in range(3):
        y = y * (1.5 - half_s * y * y)
    return jnp.minimum(y, jnp.float32(1e12))


@functools.cache
def _make_sc_kernel():
    mesh = plsc.VectorSubcoreMesh(
        core_axis_name="c", subcore_axis_name="s", num_cores=_NC
    )

    @functools.partial(
        pl.kernel,
        mesh=mesh,
        out_type=jax.ShapeDtypeStruct((4, N_CAMS * 4), jnp.float32),
        scratch_types=[
            pltpu.VMEM((9, N_VIEWS), jnp.float32),
            pltpu.VMEM((12, N_VIEWS), jnp.float32),
            pltpu.VMEM((_CPW,), jnp.int32),
            pltpu.VMEM((4, _CPW * 4), jnp.float32),
            pltpu.VMEM((4, _CPW * 4), jnp.float32),
        ],
        compiler_params=pltpu.CompilerParams(
            use_tc_tiling_on_sc=False, needs_layout_passes=False
        ),
    )
    def _sc_kernel(emb_hbm, vid_hbm, cam_hbm, out_hbm,
                   emb_v, tab_v, vid_v, cam_v, out_v):
        wid = lax.axis_index("s") * _NC + lax.axis_index("c")
        row_len = _CPW * 4

        pltpu.sync_copy(emb_hbm, emb_v)
        pltpu.sync_copy(vid_hbm.at[pl.ds(wid * _CPW, _CPW)], vid_v)
        pltpu.sync_copy(cam_hbm.at[:, pl.ds(wid * row_len, row_len)], cam_v)

        @plsc.parallel_loop(0, N_VIEWS, _L)
        def _table_body(v0):
            sl = pl.ds(v0, _L)
            dx0 = emb_v[0, sl]
            dx1 = emb_v[1, sl]
            dx2 = emb_v[2, sl]
            a1x = emb_v[3, sl] + 1.0
            a1y = emb_v[4, sl]
            a1z = emb_v[5, sl]
            a2x = emb_v[6, sl]
            a2y = emb_v[7, sl] + 1.0
            a2z = emb_v[8, sl]
            inv1 = _rsqrt16(a1x * a1x + a1y * a1y + a1z * a1z)
            b1x = a1x * inv1
            b1y = a1y * inv1
            b1z = a1z * inv1
            d = b1x * a2x + b1y * a2y + b1z * a2z
            c2x = a2x - d * b1x
            c2y = a2y - d * b1y
            c2z = a2z - d * b1z
            inv2 = _rsqrt16(c2x * c2x + c2y * c2y + c2z * c2z)
            b2x = c2x * inv2
            b2y = c2y * inv2
            b2z = c2z * inv2
            b3x = b1y * b2z - b1z * b2y
            b3y = b1z * b2x - b1x * b2z
            b3z = b1x * b2y - b1y * b2x
            comps = (b1x, b1y, b1z, dx0,
                     b2x, b2y, b2z, dx1,
                     b3x, b3y, b3z, dx2)
            for j, v in enumerate(comps):
                tab_v[j, sl] = v

        jsplat = [jnp.full((_L,), j, jnp.int32) for j in range(12)]

        @plsc.parallel_loop(0, _GROUPS, 1)
        def _main_body(g):
            ti = g // 8
            sub = g % 8
            vids = vid_v[pl.ds(g * _L, _L)]
            gcomp = [
                plsc.load_gather(tab_v, [jsplat[j], vids]) for j in range(12)
            ]
            lane0 = ti * 512 + sub * _L

            def rbody(r, carry):
                a = [cam_v[r, pl.ds(lane0 + k * 128, _L)] for k in range(4)]
                for c in range(4):
                    acc = a[0] * gcomp[c]
                    acc = acc + a[1] * gcomp[4 + c]
                    acc = acc + a[2] * gcomp[8 + c]
                    if c == 3:
                        acc = acc + a[3]
                    out_v[r, pl.ds(lane0 + c * 128, _L)] = acc
                return carry

            lax.fori_loop(0, 4, rbody, 0)

        pltpu.sync_copy(out_v, out_hbm.at[:, pl.ds(wid * row_len, row_len)])

    return _sc_kernel


def kernel(camtoworlds, view_ids, embeds):
    cam_lin = jnp.transpose(
        camtoworlds.reshape(128, 128, 4, 4), (2, 0, 3, 1)
    ).reshape(4, N_CAMS * 4)
    out_lin = _make_sc_kernel()(
        embeds.T, view_ids.astype(jnp.int32), cam_lin
    )
    return jnp.transpose(
        out_lin.reshape(4, 128, 4, 128), (1, 3, 0, 2)
    ).reshape(N_CAMS, 4, 4)


# speedup vs baseline: 1.0474x; 1.0474x over previous
import functools

import jax
import jax.numpy as jnp
from jax import lax
from jax.experimental import pallas as pl
from jax.experimental.pallas import tpu as pltpu
from jax.experimental.pallas import tpu_sc as plsc

N_CAMS = 16384
N_VIEWS = 256
_NC = 2
_NS = 16
_NW = _NC * _NS
_CPW = N_CAMS // _NW
_L = 16
_GROUPS = _CPW // _L


def _rsqrt16(s):
    i = lax.bitcast_convert_type(s, jnp.int32)
    i = jnp.int32(0x5F3759DF) - lax.shift_right_arithmetic(i, 1)
    y = lax.bitcast_convert_type(i, jnp.float32)
    half_s = 0.5 * s
    for _ in range(3):
        y = y * (1.5 - half_s * y * y)
    return jnp.minimum(y, jnp.float32(1e12))


@functools.cache
def _make_sc_kernel():
    mesh = plsc.VectorSubcoreMesh(
        core_axis_name="c", subcore_axis_name="s", num_cores=_NC
    )

    @functools.partial(
        pl.kernel,
        mesh=mesh,
        out_type=jax.ShapeDtypeStruct((4, N_CAMS * 4), jnp.float32),
        scratch_types=[
            pltpu.VMEM((9, N_VIEWS), jnp.float32),
            pltpu.VMEM((12, N_VIEWS), jnp.float32),
            pltpu.VMEM((_CPW,), jnp.int32),
            pltpu.VMEM((4, _CPW * 4), jnp.float32),
            pltpu.VMEM((4, _CPW * 4), jnp.float32),
            pltpu.SemaphoreType.DMA,
            pltpu.SemaphoreType.DMA,
        ],
        compiler_params=pltpu.CompilerParams(
            use_tc_tiling_on_sc=False, needs_layout_passes=False
        ),
    )
    def _sc_kernel(emb_hbm, vid_hbm, cam_hbm, out_hbm,
                   emb_v, tab_v, vid_v, cam_v, out_v, sem_e, sem_vc):
        wid = lax.axis_index("s") * _NC + lax.axis_index("c")
        row_len = _CPW * 4

        emb_cp = pltpu.async_copy(emb_hbm, emb_v, sem_e)
        vid_cp = pltpu.async_copy(
            vid_hbm.at[pl.ds(wid * _CPW, _CPW)], vid_v, sem_vc
        )
        cam_cp = pltpu.async_copy(
            cam_hbm.at[:, pl.ds(wid * row_len, row_len)], cam_v, sem_vc
        )
        emb_cp.wait()

        @plsc.parallel_loop(0, N_VIEWS, _L)
        def _table_body(v0):
            sl = pl.ds(v0, _L)
            dx0 = emb_v[0, sl]
            dx1 = emb_v[1, sl]
            dx2 = emb_v[2, sl]
            a1x = emb_v[3, sl] + 1.0
            a1y = emb_v[4, sl]
            a1z = emb_v[5, sl]
            a2x = emb_v[6, sl]
            a2y = emb_v[7, sl] + 1.0
            a2z = emb_v[8, sl]
            inv1 = _rsqrt16(a1x * a1x + a1y * a1y + a1z * a1z)
            b1x = a1x * inv1
            b1y = a1y * inv1
            b1z = a1z * inv1
            d = b1x * a2x + b1y * a2y + b1z * a2z
            c2x = a2x - d * b1x
            c2y = a2y - d * b1y
            c2z = a2z - d * b1z
            inv2 = _rsqrt16(c2x * c2x + c2y * c2y + c2z * c2z)
            b2x = c2x * inv2
            b2y = c2y * inv2
            b2z = c2z * inv2
            b3x = b1y * b2z - b1z * b2y
            b3y = b1z * b2x - b1x * b2z
            b3z = b1x * b2y - b1y * b2x
            comps = (b1x, b1y, b1z, dx0,
                     b2x, b2y, b2z, dx1,
                     b3x, b3y, b3z, dx2)
            for j, v in enumerate(comps):
                tab_v[j, sl] = v

        vid_cp.wait()
        cam_cp.wait()
        jsplat = [jnp.full((_L,), j, jnp.int32) for j in range(12)]

        @plsc.parallel_loop(0, _GROUPS, 1)
        def _main_body(g):
            ti = g // 8
            sub = g % 8
            vids = vid_v[pl.ds(g * _L, _L)]
            gcomp = [
                plsc.load_gather(tab_v, [jsplat[j], vids]) for j in range(12)
            ]
            lane0 = ti * 512 + sub * _L

            def rbody(r, carry):
                a = [cam_v[r, pl.ds(lane0 + k * 128, _L)] for k in range(4)]
                for c in range(4):
                    acc = a[0] * gcomp[c]
                    acc = acc + a[1] * gcomp[4 + c]
                    acc = acc + a[2] * gcomp[8 + c]
                    if c == 3:
                        acc = acc + a[3]
                    out_v[r, pl.ds(lane0 + c * 128, _L)] = acc
                return carry

            lax.fori_loop(0, 4, rbody, 0)

        pltpu.sync_copy(out_v, out_hbm.at[:, pl.ds(wid * row_len, row_len)])

    return _sc_kernel


def kernel(camtoworlds, view_ids, embeds):
    cam_lin = jnp.transpose(
        camtoworlds.reshape(128, 128, 4, 4), (2, 0, 3, 1)
    ).reshape(4, N_CAMS * 4)
    out_lin = _make_sc_kernel()(
        embeds.T, view_ids.astype(jnp.int32), cam_lin
    )
    return jnp.transpose(
        out_lin.reshape(4, 128, 4, 128), (1, 3, 0, 2)
    ).reshape(N_CAMS, 4, 4)


# speedup vs baseline: 1.1198x; 1.0691x over previous
import functools

import jax
import jax.numpy as jnp
from jax import lax
from jax.experimental import pallas as pl
from jax.experimental.pallas import tpu as pltpu
from jax.experimental.pallas import tpu_sc as plsc

N_CAMS = 16384
N_VIEWS = 256
_NC = 2
_NS = 16
_NW = _NC * _NS
_CPW = N_CAMS // _NW
_L = 16
_GROUPS = _CPW // _L


def _rsqrt16(s):
    i = lax.bitcast_convert_type(s, jnp.int32)
    i = jnp.int32(0x5F3759DF) - lax.shift_right_arithmetic(i, 1)
    y = lax.bitcast_convert_type(i, jnp.float32)
    half_s = 0.5 * s
    for _ in range(3):
        y = y * (1.5 - half_s * y * y)
    return jnp.minimum(y, jnp.float32(1e12))


@functools.cache
def _make_sc_kernel():
    mesh = plsc.VectorSubcoreMesh(
        core_axis_name="c", subcore_axis_name="s", num_cores=_NC
    )

    @functools.partial(
        pl.kernel,
        mesh=mesh,
        out_type=jax.ShapeDtypeStruct((4, N_CAMS * 4), jnp.float32),
        scratch_types=[
            pltpu.VMEM((9, _L), jnp.float32),
            pltpu.VMEM((12, N_VIEWS), jnp.float32),
            pltpu.VMEM((_CPW,), jnp.int32),
            pltpu.VMEM((4, _CPW * 4), jnp.float32),
            pltpu.VMEM((4, _CPW * 4), jnp.float32),
            pltpu.VMEM((12, _L), jnp.float32),
            pltpu.VMEM_SHARED((12, N_VIEWS), jnp.float32),
            pltpu.SemaphoreType.DMA,
            pltpu.SemaphoreType.DMA,
        ],
        compiler_params=pltpu.CompilerParams(
            use_tc_tiling_on_sc=False, needs_layout_passes=False
        ),
    )
    def _sc_kernel(emb_hbm, vid_hbm, cam_hbm, out_hbm,
                   emb_v, tab_v, vid_v, cam_v, out_v, tloc_v, tab_sh,
                   sem_e, sem_vc):
        sid = lax.axis_index("s")
        wid = sid * _NC + lax.axis_index("c")
        row_len = _CPW * 4

        emb_cp = pltpu.async_copy(
            emb_hbm.at[:, pl.ds(sid * _L, _L)], emb_v, sem_e
        )
        vid_cp = pltpu.async_copy(
            vid_hbm.at[pl.ds(wid * _CPW, _CPW)], vid_v, sem_vc
        )
        cam_cp = pltpu.async_copy(
            cam_hbm.at[:, pl.ds(wid * row_len, row_len)], cam_v, sem_vc
        )
        emb_cp.wait()

        dx0 = emb_v[0, :]
        dx1 = emb_v[1, :]
        dx2 = emb_v[2, :]
        a1x = emb_v[3, :] + 1.0
        a1y = emb_v[4, :]
        a1z = emb_v[5, :]
        a2x = emb_v[6, :]
        a2y = emb_v[7, :] + 1.0
        a2z = emb_v[8, :]
        inv1 = _rsqrt16(a1x * a1x + a1y * a1y + a1z * a1z)
        b1x = a1x * inv1
        b1y = a1y * inv1
        b1z = a1z * inv1
        d = b1x * a2x + b1y * a2y + b1z * a2z
        c2x = a2x - d * b1x
        c2y = a2y - d * b1y
        c2z = a2z - d * b1z
        inv2 = _rsqrt16(c2x * c2x + c2y * c2y + c2z * c2z)
        b2x = c2x * inv2
        b2y = c2y * inv2
        b2z = c2z * inv2
        b3x = b1y * b2z - b1z * b2y
        b3y = b1z * b2x - b1x * b2z
        b3z = b1x * b2y - b1y * b2x
        comps = (b1x, b1y, b1z, dx0,
                 b2x, b2y, b2z, dx1,
                 b3x, b3y, b3z, dx2)
        for j, v in enumerate(comps):
            tloc_v[j, :] = v
        pltpu.sync_copy(tloc_v, tab_sh.at[:, pl.ds(sid * _L, _L)])
        plsc.subcore_barrier()
        pltpu.sync_copy(tab_sh, tab_v)

        vid_cp.wait()
        cam_cp.wait()
        jsplat = [jnp.full((_L,), j, jnp.int32) for j in range(12)]

        @plsc.parallel_loop(0, _GROUPS, 1)
        def _main_body(g):
            ti = g // 8
            sub = g % 8
            vids = vid_v[pl.ds(g * _L, _L)]
            gcomp = [
                plsc.load_gather(tab_v, [jsplat[j], vids]) for j in range(12)
            ]
            lane0 = ti * 512 + sub * _L

            def rbody(r, carry):
                a = [cam_v[r, pl.ds(lane0 + k * 128, _L)] for k in range(4)]
                for c in range(4):
                    acc = a[0] * gcomp[c]
                    acc = acc + a[1] * gcomp[4 + c]
                    acc = acc + a[2] * gcomp[8 + c]
                    if c == 3:
                        acc = acc + a[3]
                    out_v[r, pl.ds(lane0 + c * 128, _L)] = acc
                return carry

            lax.fori_loop(0, 4, rbody, 0)

        pltpu.sync_copy(out_v, out_hbm.at[:, pl.ds(wid * row_len, row_len)])

    return _sc_kernel


def kernel(camtoworlds, view_ids, embeds):
    cam_lin = jnp.transpose(
        camtoworlds.reshape(128, 128, 4, 4), (2, 0, 3, 1)
    ).reshape(4, N_CAMS * 4)
    out_lin = _make_sc_kernel()(
        embeds.T, view_ids.astype(jnp.int32), cam_lin
    )
    return jnp.transpose(
        out_lin.reshape(4, 128, 4, 128), (1, 3, 0, 2)
    ).reshape(N_CAMS, 4, 4)
